# Initial kernel scaffold; baseline (speedup 1.0000x reference)
import jax
import jax.numpy as jnp
from jax.experimental import pallas as pl


def kernel(mi_gua, mi_cos, mi_fun, di_gua, di_cos, di_sem, x_m, x_d, Wx1f, bx1f, Wx2f, bx2f, Wx1s, bx1s, Wx2s, bx2s, Wy1f, by1f, Wy2f, by2f, Wy1s, by1s, Wy2s, by2s, fc1x_W, fc1x_b, fc2x_W, fc2x_b, fc1y_W, fc1y_b, fc2y_W, fc2y_b, cnnx_w, cnnx_b, cnny_w, cnny_b, lin_W, lin_b, mi_gua_edges, mi_cos_edges, mi_fun_edges, di_gua_edges, di_cos_edges, di_sem_edges, train_edges):
    raise NotImplementedError("write your pallas kernel here")


# speedup vs baseline: 28.0329x; 28.0329x over previous
import functools

import jax
import jax.numpy as jnp
from jax import lax
from jax.experimental import pallas as pl
from jax.experimental.pallas import tpu as pltpu
from jax.experimental.pallas import tpu_sc as plsc

FM = 128
NM = 215
ND = 110
EM = 8192
ED = 4096
P = 16384
NM2P = 46336
ND2P = 12160
GP = 23680

_MESH = plsc.VectorSubcoreMesh(core_axis_name="c", subcore_axis_name="s",
                               num_cores=2, num_subcores=16)


@functools.partial(
    pl.kernel,
    out_type=[
        jax.ShapeDtypeStruct((3 * NM2P,), jnp.float32),
        jax.ShapeDtypeStruct((3 * ND2P,), jnp.float32),
    ],
    mesh=_MESH,
    scratch_types=[
        pltpu.VMEM_SHARED((NM2P,), jnp.float32),
        pltpu.VMEM_SHARED((NM2P,), jnp.float32),
        pltpu.VMEM_SHARED((NM2P,), jnp.float32),
        pltpu.VMEM((NM2P // 16,), jnp.float32),
        pltpu.VMEM((128,), jnp.float32),
        pltpu.VMEM((EM // 16,), jnp.int32),
        pltpu.VMEM((EM // 16,), jnp.int32),
        pltpu.VMEM((4, 128), jnp.int32),
    ],
)
def _sc_counts(em_hbm, ed_hbm, outm_hbm, outd_hbm, c0, c1, c2, zb, ones_v,
               e0_v, e1_v, sidx_v):
    cid = lax.axis_index("c")
    sid = lax.axis_index("s")
    cbufs = [c0, c1, c2]

    zslice = NM2P // 16
    for t in range(zslice // 16):
        zb[pl.ds(t * 16, 16)] = jnp.zeros((16,), jnp.float32)
    for t in range(8):
        ones_v[pl.ds(t * 16, 16)] = jnp.ones((16,), jnp.float32)
    for k in range(3):
        pltpu.sync_copy(zb, cbufs[k].at[pl.ds(sid * zslice, zslice)])
    plsc.subcore_barrier()

    def scatter_side(e_hbm, E, N):
        epp = E // 16
        nchunks = epp // 128
        base = sid * epp
        for k in range(3):
            pltpu.sync_copy(e_hbm.at[pl.ds(k * 2 * E + base, epp)],
                            e0_v.at[pl.ds(0, epp)])
            pltpu.sync_copy(e_hbm.at[pl.ds(k * 2 * E + E + base, epp)],
                            e1_v.at[pl.ds(0, epp)])
            for v in range(epp // 16):
                i0 = e0_v[pl.ds(v * 16, 16)]
                i1 = e1_v[pl.ds(v * 16, 16)]
                sidx_v[v // 8, pl.ds((v % 8) * 16, 16)] = i0 * N + i1
            for j in range(nchunks):
                pltpu.sync_copy(ones_v, cbufs[k].at[sidx_v.at[j]], add=True)

    @pl.when(cid == 0)
    def _():
        scatter_side(em_hbm, EM, NM)

    @pl.when(cid == 1)
    def _():
        scatter_side(ed_hbm, ED, ND)

    plsc.subcore_barrier()

    @pl.when(cid == 0)
    def _():
        for k in range(3):
            pltpu.sync_copy(cbufs[k].at[pl.ds(sid * zslice, zslice)], zb)
            pltpu.sync_copy(
                zb, outm_hbm.at[pl.ds(k * NM2P + sid * zslice, zslice)])

    @pl.when(cid == 1)
    def _():
        dslice = ND2P // 16
        for k in range(3):
            pltpu.sync_copy(cbufs[k].at[pl.ds(sid * dslice, dslice)],
                            zb.at[pl.ds(0, dslice)])
            pltpu.sync_copy(
                zb.at[pl.ds(0, dslice)],
                outd_hbm.at[pl.ds(k * ND2P + sid * dslice, dslice)])


def _tc_dense_body(Sm, Cm, Sd, Cd, xm, xd,
                   Wx1f, bx1f, Wx2f, bx2f, Wx1s, bx1s, Wx2s, bx2s,
                   Wy1f, by1f, Wy2f, by2f, Wy1s, by1s, Wy2s, by2s,
                   fc1xW, fc1xb, fc2xW, fc2xb, fc1yW, fc1yb, fc2yW, fc2yb,
                   cnnxw, cnnxb, cnnyw, cnnyb, linW, linb, G_out):
    f32 = jnp.float32

    def dot(a, b, dims):
        return lax.dot_general(a, b, (dims, ((), ())),
                               preferred_element_type=f32)

    def side(x, S_ref, C_ref, Ws, N):
        W1f, b1f, W2f, b2f, W1s, b1s, W2s, b2s = Ws
        ones = jnp.full((N, 1), 1.0, f32)
        Ms, diss = [], []
        for k in range(3):
            M = C_ref[k] * S_ref[k]
            deg = dot(M, ones, (((0,), (0,)))) + 1.0
            diss.append(lax.rsqrt(deg))
            Ms.append(M)

        def layer(h_in, W, b, M, dis):
            h = dot(h_in, W, ((1,), (1,)))
            v = dis * h
            agg = dot(M, v, ((0,), (0,)))
            out = dis * agg + (dis * dis) * h + b
            return jnp.maximum(out, 0.0)

        f1 = layer(x, W1f, b1f, Ms[0], diss[0])
        f2 = layer(f1, W2f, b2f, Ms[0], diss[0])
        s1 = layer(x, W1s, b1s, Ms[1], diss[1])
        s2 = layer(s1, W2s, b2s, Ms[1], diss[1])
        g1 = layer(x, W1s, b1s, Ms[2], diss[2])
        g2 = layer(s1, W2s, b2s, Ms[2], diss[2])
        return [f1, f2, s1, s2, g1, g2]

    def atten(feats, fc1W, fc1b, fc2W, fc2b, cw, cb, N):
        scale = 1.0 / (N * FM)
        a = jnp.concatenate(
            [(jnp.sum(f) * scale).reshape(1, 1) for f in feats], axis=1)
        a = jnp.maximum(dot(a, fc1W[...], ((1,), (1,))) + fc1b[...], 0.0)
        a = jax.nn.sigmoid(dot(a, fc2W[...], ((1,), (1,))) + fc2b[...])
        cwv = cw[...]
        out = cb[0, 0]
        for c in range(6):
            out = out + cwv[0, c] * jnp.maximum(a[0, c] * feats[c], 0.0)
        return out

    xw = (Wx1f[...], bx1f[...], Wx2f[...], bx2f[...],
          Wx1s[...], bx1s[...], Wx2s[...], bx2s[...])
    yw = (Wy1f[...], by1f[...], Wy2f[...], by2f[...],
          Wy1s[...], by1s[...], Wy2s[...], by2s[...])
    mf = side(xm[...], Sm, Cm, xw, NM)
    df = side(xd[...], Sd, Cd, yw, ND)
    x = atten(mf, fc1xW, fc1xb, fc2xW, fc2xb, cnnxw, cnnxb, NM)
    y = atten(df, fc1yW, fc1yb, fc2yW, fc2yb, cnnyw, cnnyb, ND)
    G_out[...] = dot(x * linW[...], y, ((1,), (1,))) + linb[0, 0]


@functools.partial(
    pl.kernel,
    out_type=jax.ShapeDtypeStruct((P,), jnp.float32),
    mesh=_MESH,
    scratch_types=[
        pltpu.VMEM((P // 32,), jnp.int32),
        pltpu.VMEM((P // 32,), jnp.int32),
        pltpu.VMEM((4, 128), jnp.int32),
        pltpu.VMEM((4, 128), jnp.float32),
        pltpu.VMEM((P // 32,), jnp.float32),
        pltpu.SemaphoreType.DMA,
    ],
)
def _sc_pair_gather(g_hbm, te_hbm, out_hbm, t0_v, t1_v, sidx_v, r_v, o_v, sem):
    wid = lax.axis_index("s") * 2 + lax.axis_index("c")
    npp = P // 32
    base = wid * npp
    pltpu.sync_copy(te_hbm.at[pl.ds(base, npp)], t0_v)
    pltpu.sync_copy(te_hbm.at[pl.ds(P + base, npp)], t1_v)
    for v in range(npp // 16):
        i0 = t0_v[pl.ds(v * 16, 16)]
        i1 = t1_v[pl.ds(v * 16, 16)]
        sidx_v[v // 8, pl.ds((v % 8) * 16, 16)] = i0 * ND + i1
    for j in range(npp // 128):
        pltpu.async_copy(g_hbm.at[sidx_v.at[j]], r_v.at[j], sem).wait()
    for v in range(npp // 16):
        g = r_v[v // 8, pl.ds((v % 8) * 16, 16)]
        o_v[pl.ds(v * 16, 16)] = 1.0 / (1.0 + jnp.exp(-g))
    pltpu.sync_copy(o_v, out_hbm.at[pl.ds(base, npp)])


def kernel(mi_gua, mi_cos, mi_fun, di_gua, di_cos, di_sem, x_m, x_d,
           Wx1f, bx1f, Wx2f, bx2f, Wx1s, bx1s, Wx2s, bx2s,
           Wy1f, by1f, Wy2f, by2f, Wy1s, by1s, Wy2s, by2s,
           fc1x_W, fc1x_b, fc2x_W, fc2x_b, fc1y_W, fc1y_b, fc2y_W, fc2y_b,
           cnnx_w, cnnx_b, cnny_w, cnny_b, lin_W, lin_b,
           mi_gua_edges, mi_cos_edges, mi_fun_edges,
           di_gua_edges, di_cos_edges, di_sem_edges, train_edges):
    f32 = jnp.float32
    em = jnp.stack([mi_gua_edges, mi_cos_edges, mi_fun_edges]).astype(
        jnp.int32).reshape(-1)
    ed = jnp.stack([di_gua_edges, di_cos_edges, di_sem_edges]).astype(
        jnp.int32).reshape(-1)

    cm_raw, cd_raw = _sc_counts(em, ed)
    Cm = cm_raw.reshape(3, NM2P)[:, :NM * NM].reshape(3, NM, NM)
    Cd = cd_raw.reshape(3, ND2P)[:, :ND * ND].reshape(3, ND, ND)
    Sm = jnp.stack([mi_gua, mi_cos, mi_fun])
    Sd = jnp.stack([di_gua, di_cos, di_sem])

    G = pl.pallas_call(
        _tc_dense_body,
        out_shape=jax.ShapeDtypeStruct((NM, ND), f32),
    )(Sm, Cm, Sd, Cd, x_m, x_d,
      Wx1f, bx1f.reshape(1, FM), Wx2f, bx2f.reshape(1, FM),
      Wx1s, bx1s.reshape(1, FM), Wx2s, bx2s.reshape(1, FM),
      Wy1f, by1f.reshape(1, FM), Wy2f, by2f.reshape(1, FM),
      Wy1s, by1s.reshape(1, FM), Wy2s, by2s.reshape(1, FM),
      fc1x_W, fc1x_b.reshape(1, 30), fc2x_W, fc2x_b.reshape(1, 6),
      fc1y_W, fc1y_b.reshape(1, 30), fc2y_W, fc2y_b.reshape(1, 6),
      cnnx_w.reshape(1, 6), cnnx_b.reshape(1, 1),
      cnny_w.reshape(1, 6), cnny_b.reshape(1, 1),
      lin_W, lin_b.reshape(1, 1))

    gp = jnp.concatenate([G.reshape(-1), jnp.zeros((GP - NM * ND,), f32)])
    te = train_edges.T.astype(jnp.int32).reshape(-1)
    return _sc_pair_gather(gp, te)


# speedup vs baseline: 30.8427x; 1.1002x over previous
import functools

import jax
import jax.numpy as jnp
from jax import lax
from jax.experimental import pallas as pl
from jax.experimental.pallas import tpu as pltpu
from jax.experimental.pallas import tpu_sc as plsc

FM = 128
NM = 215
ND = 110
EM = 8192
ED = 4096
P = 16384
NM2P = 46336
ND2P = 12160
GP = 23680

_MESH = plsc.VectorSubcoreMesh(core_axis_name="c", subcore_axis_name="s",
                               num_cores=2, num_subcores=16)


@functools.partial(
    pl.kernel,
    out_type=[
        jax.ShapeDtypeStruct((3 * NM2P,), jnp.float32),
        jax.ShapeDtypeStruct((3 * ND2P,), jnp.float32),
    ],
    mesh=_MESH,
    scratch_types=[
        pltpu.VMEM_SHARED((NM2P,), jnp.float32),
        pltpu.VMEM_SHARED((NM2P,), jnp.float32),
        pltpu.VMEM_SHARED((NM2P,), jnp.float32),
        pltpu.VMEM((NM2P // 16,), jnp.float32),
        pltpu.VMEM((128,), jnp.float32),
        pltpu.VMEM((EM // 16,), jnp.int32),
        pltpu.VMEM((EM // 16,), jnp.int32),
        pltpu.VMEM((EM // 16,), jnp.int32),
        pltpu.VMEM((EM // 16,), jnp.int32),
        pltpu.VMEM((EM // 16,), jnp.int32),
        pltpu.VMEM((EM // 16,), jnp.int32),
        pltpu.VMEM((12, 128), jnp.int32),
        pltpu.SemaphoreType.DMA,
    ],
)
def _sc_counts(em_hbm, ed_hbm, outm_hbm, outd_hbm, c0, c1, c2, zb, ones_v,
               e0a, e0b, e1a, e1b, e2a, e2b, sidx_v, sem):
    cid = lax.axis_index("c")
    sid = lax.axis_index("s")
    cbufs = [c0, c1, c2]
    ebufs = [(e0a, e0b), (e1a, e1b), (e2a, e2b)]
    zslice = NM2P // 16

    def scatter_side(e_hbm, E, N):
        epp = E // 16
        nchunks = epp // 128
        base = sid * epp

        in_dmas = []
        for k in range(3):
            in_dmas.append(pltpu.async_copy(
                e_hbm.at[pl.ds(k * 2 * E + base, epp)],
                ebufs[k][0].at[pl.ds(0, epp)], sem))
            in_dmas.append(pltpu.async_copy(
                e_hbm.at[pl.ds(k * 2 * E + E + base, epp)],
                ebufs[k][1].at[pl.ds(0, epp)], sem))

        for t in range(zslice // 16):
            zb[pl.ds(t * 16, 16)] = jnp.zeros((16,), jnp.float32)
        for t in range(8):
            ones_v[pl.ds(t * 16, 16)] = jnp.ones((16,), jnp.float32)
        for k in range(3):
            pltpu.sync_copy(zb, cbufs[k].at[pl.ds(sid * zslice, zslice)])

        for d in in_dmas:
            d.wait()
        for k in range(3):
            for v in range(epp // 16):
                i0 = ebufs[k][0][pl.ds(v * 16, 16)]
                i1 = ebufs[k][1][pl.ds(v * 16, 16)]
                sidx_v[k * nchunks + v // 8,
                       pl.ds((v % 8) * 16, 16)] = i0 * N + i1
        plsc.subcore_barrier()

        for k in range(3):
            for j in range(nchunks):
                pltpu.sync_copy(ones_v,
                                cbufs[k].at[sidx_v.at[k * nchunks + j]],
                                add=True)

    @pl.when(cid == 0)
    def _():
        scatter_side(em_hbm, EM, NM)

    @pl.when(cid == 1)
    def _():
        scatter_side(ed_hbm, ED, ND)

    plsc.subcore_barrier()

    @pl.when(cid == 0)
    def _():
        for k in range(3):
            pltpu.sync_copy(cbufs[k].at[pl.ds(sid * zslice, zslice)], zb)
            pltpu.sync_copy(
                zb, outm_hbm.at[pl.ds(k * NM2P + sid * zslice, zslice)])

    @pl.when(cid == 1)
    def _():
        dslice = ND2P // 16
        for k in range(3):
            pltpu.sync_copy(cbufs[k].at[pl.ds(sid * dslice, dslice)],
                            zb.at[pl.ds(0, dslice)])
            pltpu.sync_copy(
                zb.at[pl.ds(0, dslice)],
                outd_hbm.at[pl.ds(k * ND2P + sid * dslice, dslice)])


def _tc_dense_body(Sm, Cm, Sd, Cd, xm, xd,
                   Wx1f, bx1f, Wx2f, bx2f, Wx1s, bx1s, Wx2s, bx2s,
                   Wy1f, by1f, Wy2f, by2f, Wy1s, by1s, Wy2s, by2s,
                   fc1xW, fc1xb, fc2xW, fc2xb, fc1yW, fc1yb, fc2yW, fc2yb,
                   cnnxw, cnnxb, cnnyw, cnnyb, linW, linb, G_out):
    f32 = jnp.float32

    def dot(a, b, dims):
        return lax.dot_general(a, b, (dims, ((), ())),
                               preferred_element_type=f32)

    def side(x, S_ref, C_ref, Ws, N):
        W1f, b1f, W2f, b2f, W1s, b1s, W2s, b2s = Ws
        ones = jnp.full((N, 1), 1.0, f32)
        Ms, diss = [], []
        for k in range(3):
            M = C_ref[k] * S_ref[k]
            deg = dot(M, ones, (((0,), (0,)))) + 1.0
            diss.append(lax.rsqrt(deg))
            Ms.append(M)

        def layer(h_in, W, b, M, dis):
            h = dot(h_in, W, ((1,), (1,)))
            v = dis * h
            agg = dot(M, v, ((0,), (0,)))
            out = dis * agg + (dis * dis) * h + b
            return jnp.maximum(out, 0.0)

        f1 = layer(x, W1f, b1f, Ms[0], diss[0])
        f2 = layer(f1, W2f, b2f, Ms[0], diss[0])
        s1 = layer(x, W1s, b1s, Ms[1], diss[1])
        s2 = layer(s1, W2s, b2s, Ms[1], diss[1])
        g1 = layer(x, W1s, b1s, Ms[2], diss[2])
        g2 = layer(s1, W2s, b2s, Ms[2], diss[2])
        return [f1, f2, s1, s2, g1, g2]

    def atten(feats, fc1W, fc1b, fc2W, fc2b, cw, cb, N):
        scale = 1.0 / (N * FM)
        a = jnp.concatenate(
            [(jnp.sum(f) * scale).reshape(1, 1) for f in feats], axis=1)
        a = jnp.maximum(dot(a, fc1W[...], ((1,), (1,))) + fc1b[...], 0.0)
        a = jax.nn.sigmoid(dot(a, fc2W[...], ((1,), (1,))) + fc2b[...])
        cwv = cw[...]
        out = cb[0, 0]
        for c in range(6):
            out = out + cwv[0, c] * jnp.maximum(a[0, c] * feats[c], 0.0)
        return out

    xw = (Wx1f[...], bx1f[...], Wx2f[...], bx2f[...],
          Wx1s[...], bx1s[...], Wx2s[...], bx2s[...])
    yw = (Wy1f[...], by1f[...], Wy2f[...], by2f[...],
          Wy1s[...], by1s[...], Wy2s[...], by2s[...])
    mf = side(xm[...], Sm, Cm, xw, NM)
    df = side(xd[...], Sd, Cd, yw, ND)
    x = atten(mf, fc1xW, fc1xb, fc2xW, fc2xb, cnnxw, cnnxb, NM)
    y = atten(df, fc1yW, fc1yb, fc2yW, fc2yb, cnnyw, cnnyb, ND)
    G_out[...] = dot(x * linW[...], y, ((1,), (1,))) + linb[0, 0]


@functools.partial(
    pl.kernel,
    out_type=jax.ShapeDtypeStruct((P,), jnp.float32),
    mesh=_MESH,
    scratch_types=[
        pltpu.VMEM((P // 32,), jnp.int32),
        pltpu.VMEM((P // 32,), jnp.int32),
        pltpu.VMEM((4, 128), jnp.int32),
        pltpu.VMEM((4, 128), jnp.float32),
        pltpu.VMEM((P // 32,), jnp.float32),
        pltpu.SemaphoreType.DMA,
        pltpu.SemaphoreType.DMA,
    ],
)
def _sc_pair_gather(g_hbm, te_hbm, out_hbm, t0_v, t1_v, sidx_v, r_v, o_v,
                    sem, sem2):
    wid = lax.axis_index("s") * 2 + lax.axis_index("c")
    npp = P // 32
    base = wid * npp
    d0 = pltpu.async_copy(te_hbm.at[pl.ds(base, npp)], t0_v, sem)
    d1 = pltpu.async_copy(te_hbm.at[pl.ds(P + base, npp)], t1_v, sem)
    d0.wait()
    d1.wait()
    for v in range(npp // 16):
        i0 = t0_v[pl.ds(v * 16, 16)]
        i1 = t1_v[pl.ds(v * 16, 16)]
        sidx_v[v // 8, pl.ds((v % 8) * 16, 16)] = i0 * ND + i1
    gd = [pltpu.async_copy(g_hbm.at[sidx_v.at[j]], r_v.at[j], sem2)
          for j in range(npp // 128)]
    for d in gd:
        d.wait()
    for v in range(npp // 16):
        g = r_v[v // 8, pl.ds((v % 8) * 16, 16)]
        o_v[pl.ds(v * 16, 16)] = 1.0 / (1.0 + jnp.exp(-g))
    pltpu.sync_copy(o_v, out_hbm.at[pl.ds(base, npp)])


def kernel(mi_gua, mi_cos, mi_fun, di_gua, di_cos, di_sem, x_m, x_d,
           Wx1f, bx1f, Wx2f, bx2f, Wx1s, bx1s, Wx2s, bx2s,
           Wy1f, by1f, Wy2f, by2f, Wy1s, by1s, Wy2s, by2s,
           fc1x_W, fc1x_b, fc2x_W, fc2x_b, fc1y_W, fc1y_b, fc2y_W, fc2y_b,
           cnnx_w, cnnx_b, cnny_w, cnny_b, lin_W, lin_b,
           mi_gua_edges, mi_cos_edges, mi_fun_edges,
           di_gua_edges, di_cos_edges, di_sem_edges, train_edges):
    f32 = jnp.float32
    em = jnp.stack([mi_gua_edges, mi_cos_edges, mi_fun_edges]).astype(
        jnp.int32).reshape(-1)
    ed = jnp.stack([di_gua_edges, di_cos_edges, di_sem_edges]).astype(
        jnp.int32).reshape(-1)

    cm_raw, cd_raw = _sc_counts(em, ed)
    Cm = cm_raw.reshape(3, NM2P)[:, :NM * NM].reshape(3, NM, NM)
    Cd = cd_raw.reshape(3, ND2P)[:, :ND * ND].reshape(3, ND, ND)
    Sm = jnp.stack([mi_gua, mi_cos, mi_fun])
    Sd = jnp.stack([di_gua, di_cos, di_sem])

    G = pl.pallas_call(
        _tc_dense_body,
        out_shape=jax.ShapeDtypeStruct((NM, ND), f32),
    )(Sm, Cm, Sd, Cd, x_m, x_d,
      Wx1f, bx1f.reshape(1, FM), Wx2f, bx2f.reshape(1, FM),
      Wx1s, bx1s.reshape(1, FM), Wx2s, bx2s.reshape(1, FM),
      Wy1f, by1f.reshape(1, FM), Wy2f, by2f.reshape(1, FM),
      Wy1s, by1s.reshape(1, FM), Wy2s, by2s.reshape(1, FM),
      fc1x_W, fc1x_b.reshape(1, 30), fc2x_W, fc2x_b.reshape(1, 6),
      fc1y_W, fc1y_b.reshape(1, 30), fc2y_W, fc2y_b.reshape(1, 6),
      cnnx_w.reshape(1, 6), cnnx_b.reshape(1, 1),
      cnny_w.reshape(1, 6), cnny_b.reshape(1, 1),
      lin_W, lin_b.reshape(1, 1))

    gp = jnp.concatenate([G.reshape(-1), jnp.zeros((GP - NM * ND,), f32)])
    te = train_edges.T.astype(jnp.int32).reshape(-1)
    return _sc_pair_gather(gp, te)


# speedup vs baseline: 30.9827x; 1.0045x over previous
import functools

import jax
import jax.numpy as jnp
from jax import lax
from jax.experimental import pallas as pl
from jax.experimental.pallas import tpu as pltpu
from jax.experimental.pallas import tpu_sc as plsc

FM = 128
NM = 215
ND = 110
EM = 8192
ED = 4096
P = 16384
NM2P = 46336
ND2P = 12160
GP = 23680

_MESH = plsc.VectorSubcoreMesh(core_axis_name="c", subcore_axis_name="s",
                               num_cores=2, num_subcores=16)


@functools.partial(
    pl.kernel,
    out_type=[
        jax.ShapeDtypeStruct((3 * NM2P,), jnp.float32),
        jax.ShapeDtypeStruct((3 * ND2P,), jnp.float32),
    ],
    mesh=_MESH,
    scratch_types=[
        pltpu.VMEM_SHARED((NM2P,), jnp.float32),
        pltpu.VMEM_SHARED((NM2P,), jnp.float32),
        pltpu.VMEM_SHARED((NM2P,), jnp.float32),
        pltpu.VMEM((NM2P // 16,), jnp.float32),
        pltpu.VMEM((NM2P // 16,), jnp.float32),
        pltpu.VMEM((NM2P // 16,), jnp.float32),
        pltpu.VMEM((128,), jnp.float32),
        pltpu.VMEM((EM // 16,), jnp.int32),
        pltpu.VMEM((EM // 16,), jnp.int32),
        pltpu.VMEM((EM // 16,), jnp.int32),
        pltpu.VMEM((EM // 16,), jnp.int32),
        pltpu.VMEM((EM // 16,), jnp.int32),
        pltpu.VMEM((EM // 16,), jnp.int32),
        pltpu.VMEM((12, 128), jnp.int32),
        pltpu.SemaphoreType.DMA,
    ],
)
def _sc_counts(em_hbm, ed_hbm, outm_hbm, outd_hbm, c0, c1, c2, zb, zb1, zb2,
               ones_v, e0a, e0b, e1a, e1b, e2a, e2b, sidx_v, sem):
    cid = lax.axis_index("c")
    sid = lax.axis_index("s")
    cbufs = [c0, c1, c2]
    ebufs = [(e0a, e0b), (e1a, e1b), (e2a, e2b)]
    ebounce = [zb, zb1, zb2]
    zslice = NM2P // 16

    def scatter_side(e_hbm, E, N):
        epp = E // 16
        nchunks = epp // 128
        base = sid * epp

        in_dmas = []
        for k in range(3):
            in_dmas.append(pltpu.async_copy(
                e_hbm.at[pl.ds(k * 2 * E + base, epp)],
                ebufs[k][0].at[pl.ds(0, epp)], sem))
            in_dmas.append(pltpu.async_copy(
                e_hbm.at[pl.ds(k * 2 * E + E + base, epp)],
                ebufs[k][1].at[pl.ds(0, epp)], sem))

        for t in range(zslice // 16):
            zb[pl.ds(t * 16, 16)] = jnp.zeros((16,), jnp.float32)
        for t in range(8):
            ones_v[pl.ds(t * 16, 16)] = jnp.ones((16,), jnp.float32)
        for k in range(3):
            pltpu.sync_copy(zb, cbufs[k].at[pl.ds(sid * zslice, zslice)])

        for d in in_dmas:
            d.wait()
        for k in range(3):
            for v in range(epp // 16):
                i0 = ebufs[k][0][pl.ds(v * 16, 16)]
                i1 = ebufs[k][1][pl.ds(v * 16, 16)]
                sidx_v[k * nchunks + v // 8,
                       pl.ds((v % 8) * 16, 16)] = i0 * N + i1
        plsc.subcore_barrier()

        for k in range(3):
            for j in range(nchunks):
                pltpu.sync_copy(ones_v,
                                cbufs[k].at[sidx_v.at[k * nchunks + j]],
                                add=True)

    @pl.when(cid == 0)
    def _():
        scatter_side(em_hbm, EM, NM)

    @pl.when(cid == 1)
    def _():
        scatter_side(ed_hbm, ED, ND)

    plsc.subcore_barrier()

    def write_out(out_hbm, OUTP):
        oslice = OUTP // 16
        out_dmas = []
        for k in range(3):
            pltpu.sync_copy(cbufs[k].at[pl.ds(sid * oslice, oslice)],
                            ebounce[k].at[pl.ds(0, oslice)])
            out_dmas.append(pltpu.async_copy(
                ebounce[k].at[pl.ds(0, oslice)],
                out_hbm.at[pl.ds(k * OUTP + sid * oslice, oslice)], sem))
        for d in out_dmas:
            d.wait()

    @pl.when(cid == 0)
    def _():
        write_out(outm_hbm, NM2P)

    @pl.when(cid == 1)
    def _():
        write_out(outd_hbm, ND2P)


def _tc_dense_body(Sm, Cm, Sd, Cd, xm, xd,
                   Wx1f, bx1f, Wx2f, bx2f, Wx1s, bx1s, Wx2s, bx2s,
                   Wy1f, by1f, Wy2f, by2f, Wy1s, by1s, Wy2s, by2s,
                   fc1xW, fc1xb, fc2xW, fc2xb, fc1yW, fc1yb, fc2yW, fc2yb,
                   cnnxw, cnnxb, cnnyw, cnnyb, linW, linb, G_out):
    f32 = jnp.float32

    def dot(a, b, dims):
        return lax.dot_general(a, b, (dims, ((), ())),
                               preferred_element_type=f32)

    def side(x, S_ref, C_ref, Ws, N):
        W1f, b1f, W2f, b2f, W1s, b1s, W2s, b2s = Ws
        ones = jnp.full((N, 1), 1.0, f32)
        Ms, diss = [], []
        for k in range(3):
            M = C_ref[k] * S_ref[k]
            deg = dot(M, ones, (((0,), (0,)))) + 1.0
            diss.append(lax.rsqrt(deg))
            Ms.append(M)

        def layer(h_in, W, b, M, dis):
            h = dot(h_in, W, ((1,), (1,)))
            v = dis * h
            agg = dot(M, v, ((0,), (0,)))
            out = dis * agg + (dis * dis) * h + b
            return jnp.maximum(out, 0.0)

        f1 = layer(x, W1f, b1f, Ms[0], diss[0])
        f2 = layer(f1, W2f, b2f, Ms[0], diss[0])
        s1 = layer(x, W1s, b1s, Ms[1], diss[1])
        s2 = layer(s1, W2s, b2s, Ms[1], diss[1])
        g1 = layer(x, W1s, b1s, Ms[2], diss[2])
        g2 = layer(s1, W2s, b2s, Ms[2], diss[2])
        return [f1, f2, s1, s2, g1, g2]

    def atten(feats, fc1W, fc1b, fc2W, fc2b, cw, cb, N):
        scale = 1.0 / (N * FM)
        a = jnp.concatenate(
            [(jnp.sum(f) * scale).reshape(1, 1) for f in feats], axis=1)
        a = jnp.maximum(dot(a, fc1W[...], ((1,), (1,))) + fc1b[...], 0.0)
        a = jax.nn.sigmoid(dot(a, fc2W[...], ((1,), (1,))) + fc2b[...])
        cwv = cw[...]
        out = cb[0, 0]
        for c in range(6):
            out = out + cwv[0, c] * jnp.maximum(a[0, c] * feats[c], 0.0)
        return out

    xw = (Wx1f[...], bx1f[...], Wx2f[...], bx2f[...],
          Wx1s[...], bx1s[...], Wx2s[...], bx2s[...])
    yw = (Wy1f[...], by1f[...], Wy2f[...], by2f[...],
          Wy1s[...], by1s[...], Wy2s[...], by2s[...])
    mf = side(xm[...], Sm, Cm, xw, NM)
    df = side(xd[...], Sd, Cd, yw, ND)
    x = atten(mf, fc1xW, fc1xb, fc2xW, fc2xb, cnnxw, cnnxb, NM)
    y = atten(df, fc1yW, fc1yb, fc2yW, fc2yb, cnnyw, cnnyb, ND)
    G_out[...] = dot(x * linW[...], y, ((1,), (1,))) + linb[0, 0]


@functools.partial(
    pl.kernel,
    out_type=jax.ShapeDtypeStruct((P,), jnp.float32),
    mesh=_MESH,
    scratch_types=[
        pltpu.VMEM((P // 32,), jnp.int32),
        pltpu.VMEM((P // 32,), jnp.int32),
        pltpu.VMEM((4, 128), jnp.int32),
        pltpu.VMEM((4, 128), jnp.float32),
        pltpu.VMEM((P // 32,), jnp.float32),
        pltpu.SemaphoreType.DMA,
        pltpu.SemaphoreType.DMA,
    ],
)
def _sc_pair_gather(g_hbm, te_hbm, out_hbm, t0_v, t1_v, sidx_v, r_v, o_v,
                    sem, sem2):
    wid = lax.axis_index("s") * 2 + lax.axis_index("c")
    npp = P // 32
    base = wid * npp
    d0 = pltpu.async_copy(te_hbm.at[pl.ds(base, npp)], t0_v, sem)
    d1 = pltpu.async_copy(te_hbm.at[pl.ds(P + base, npp)], t1_v, sem)
    d0.wait()
    d1.wait()
    for v in range(npp // 16):
        i0 = t0_v[pl.ds(v * 16, 16)]
        i1 = t1_v[pl.ds(v * 16, 16)]
        sidx_v[v // 8, pl.ds((v % 8) * 16, 16)] = i0 * ND + i1
    gd = [pltpu.async_copy(g_hbm.at[sidx_v.at[j]], r_v.at[j], sem2)
          for j in range(npp // 128)]
    for d in gd:
        d.wait()
    for v in range(npp // 16):
        g = r_v[v // 8, pl.ds((v % 8) * 16, 16)]
        o_v[pl.ds(v * 16, 16)] = 1.0 / (1.0 + jnp.exp(-g))
    pltpu.sync_copy(o_v, out_hbm.at[pl.ds(base, npp)])


def kernel(mi_gua, mi_cos, mi_fun, di_gua, di_cos, di_sem, x_m, x_d,
           Wx1f, bx1f, Wx2f, bx2f, Wx1s, bx1s, Wx2s, bx2s,
           Wy1f, by1f, Wy2f, by2f, Wy1s, by1s, Wy2s, by2s,
           fc1x_W, fc1x_b, fc2x_W, fc2x_b, fc1y_W, fc1y_b, fc2y_W, fc2y_b,
           cnnx_w, cnnx_b, cnny_w, cnny_b, lin_W, lin_b,
           mi_gua_edges, mi_cos_edges, mi_fun_edges,
           di_gua_edges, di_cos_edges, di_sem_edges, train_edges):
    f32 = jnp.float32
    em = jnp.stack([mi_gua_edges, mi_cos_edges, mi_fun_edges]).astype(
        jnp.int32).reshape(-1)
    ed = jnp.stack([di_gua_edges, di_cos_edges, di_sem_edges]).astype(
        jnp.int32).reshape(-1)

    cm_raw, cd_raw = _sc_counts(em, ed)
    Cm = cm_raw.reshape(3, NM2P)[:, :NM * NM].reshape(3, NM, NM)
    Cd = cd_raw.reshape(3, ND2P)[:, :ND * ND].reshape(3, ND, ND)
    Sm = jnp.stack([mi_gua, mi_cos, mi_fun])
    Sd = jnp.stack([di_gua, di_cos, di_sem])

    G = pl.pallas_call(
        _tc_dense_body,
        out_shape=jax.ShapeDtypeStruct((NM, ND), f32),
    )(Sm, Cm, Sd, Cd, x_m, x_d,
      Wx1f, bx1f.reshape(1, FM), Wx2f, bx2f.reshape(1, FM),
      Wx1s, bx1s.reshape(1, FM), Wx2s, bx2s.reshape(1, FM),
      Wy1f, by1f.reshape(1, FM), Wy2f, by2f.reshape(1, FM),
      Wy1s, by1s.reshape(1, FM), Wy2s, by2s.reshape(1, FM),
      fc1x_W, fc1x_b.reshape(1, 30), fc2x_W, fc2x_b.reshape(1, 6),
      fc1y_W, fc1y_b.reshape(1, 30), fc2y_W, fc2y_b.reshape(1, 6),
      cnnx_w.reshape(1, 6), cnnx_b.reshape(1, 1),
      cnny_w.reshape(1, 6), cnny_b.reshape(1, 1),
      lin_W, lin_b.reshape(1, 1))

    gp = jnp.concatenate([G.reshape(-1), jnp.zeros((GP - NM * ND,), f32)])
    te = train_edges.T.astype(jnp.int32).reshape(-1)
    return _sc_pair_gather(gp, te)


# speedup vs baseline: 31.0402x; 1.0019x over previous
import functools

import jax
import jax.numpy as jnp
from jax import lax
from jax.experimental import pallas as pl
from jax.experimental.pallas import tpu as pltpu
from jax.experimental.pallas import tpu_sc as plsc

FM = 128
NM = 215
ND = 110
EM = 8192
ED = 4096
P = 16384
NM2P = 46336
ND2P = 12160
GR = 216
GC = 128
GP = GR * GC

_MESH = plsc.VectorSubcoreMesh(core_axis_name="c", subcore_axis_name="s",
                               num_cores=2, num_subcores=16)


@functools.partial(
    pl.kernel,
    out_type=[
        jax.ShapeDtypeStruct((3 * NM2P,), jnp.float32),
        jax.ShapeDtypeStruct((3 * ND2P,), jnp.float32),
    ],
    mesh=_MESH,
    scratch_types=[
        pltpu.VMEM_SHARED((NM2P,), jnp.float32),
        pltpu.VMEM_SHARED((NM2P,), jnp.float32),
        pltpu.VMEM_SHARED((NM2P,), jnp.float32),
        pltpu.VMEM((NM2P // 16,), jnp.float32),
        pltpu.VMEM((NM2P // 16,), jnp.float32),
        pltpu.VMEM((NM2P // 16,), jnp.float32),
        pltpu.VMEM((128,), jnp.float32),
        pltpu.VMEM((EM // 16,), jnp.int32),
        pltpu.VMEM((EM // 16,), jnp.int32),
        pltpu.VMEM((EM // 16,), jnp.int32),
        pltpu.VMEM((EM // 16,), jnp.int32),
        pltpu.VMEM((EM // 16,), jnp.int32),
        pltpu.VMEM((EM // 16,), jnp.int32),
        pltpu.VMEM((12, 128), jnp.int32),
        pltpu.SemaphoreType.DMA,
    ],
)
def _sc_counts(em0_hbm, em1_hbm, em2_hbm, ed0_hbm, ed1_hbm, ed2_hbm,
               outm_hbm, outd_hbm, c0, c1, c2, zb, zb1, zb2,
               ones_v, e0a, e0b, e1a, e1b, e2a, e2b, sidx_v, sem):
    cid = lax.axis_index("c")
    sid = lax.axis_index("s")
    cbufs = [c0, c1, c2]
    ebufs = [(e0a, e0b), (e1a, e1b), (e2a, e2b)]
    ebounce = [zb, zb1, zb2]
    zslice = NM2P // 16

    def scatter_side(e_hbms, E, N):
        epp = E // 16
        nchunks = epp // 128
        base = sid * epp

        in_dmas = []
        for k in range(3):
            in_dmas.append(pltpu.async_copy(
                e_hbms[k].at[pl.ds(base, epp)],
                ebufs[k][0].at[pl.ds(0, epp)], sem))
            in_dmas.append(pltpu.async_copy(
                e_hbms[k].at[pl.ds(E + base, epp)],
                ebufs[k][1].at[pl.ds(0, epp)], sem))

        for t in range(zslice // 16):
            zb[pl.ds(t * 16, 16)] = jnp.zeros((16,), jnp.float32)
        for t in range(8):
            ones_v[pl.ds(t * 16, 16)] = jnp.ones((16,), jnp.float32)
        for k in range(3):
            pltpu.sync_copy(zb, cbufs[k].at[pl.ds(sid * zslice, zslice)])

        for d in in_dmas:
            d.wait()
        for k in range(3):
            for v in range(epp // 16):
                i0 = ebufs[k][0][pl.ds(v * 16, 16)]
                i1 = ebufs[k][1][pl.ds(v * 16, 16)]
                sidx_v[k * nchunks + v // 8,
                       pl.ds((v % 8) * 16, 16)] = i0 * N + i1
        plsc.subcore_barrier()

        for k in range(3):
            for j in range(nchunks):
                pltpu.sync_copy(ones_v,
                                cbufs[k].at[sidx_v.at[k * nchunks + j]],
                                add=True)

    @pl.when(cid == 0)
    def _():
        scatter_side([em0_hbm, em1_hbm, em2_hbm], EM, NM)

    @pl.when(cid == 1)
    def _():
        scatter_side([ed0_hbm, ed1_hbm, ed2_hbm], ED, ND)

    plsc.subcore_barrier()

    def write_out(out_hbm, OUTP):
        oslice = OUTP // 16
        out_dmas = []
        for k in range(3):
            pltpu.sync_copy(cbufs[k].at[pl.ds(sid * oslice, oslice)],
                            ebounce[k].at[pl.ds(0, oslice)])
            out_dmas.append(pltpu.async_copy(
                ebounce[k].at[pl.ds(0, oslice)],
                out_hbm.at[pl.ds(k * OUTP + sid * oslice, oslice)], sem))
        for d in out_dmas:
            d.wait()

    @pl.when(cid == 0)
    def _():
        write_out(outm_hbm, NM2P)

    @pl.when(cid == 1)
    def _():
        write_out(outd_hbm, ND2P)


def _tc_dense_body(Sm0, Sm1, Sm2, Cm, Sd0, Sd1, Sd2, Cd, xm, xd,
                   Wx1f, bx1f, Wx2f, bx2f, Wx1s, bx1s, Wx2s, bx2s,
                   Wy1f, by1f, Wy2f, by2f, Wy1s, by1s, Wy2s, by2s,
                   fc1xW, fc1xb, fc2xW, fc2xb, fc1yW, fc1yb, fc2yW, fc2yb,
                   cnnxw, cnnxb, cnnyw, cnnyb, linW, linb, G_out):
    f32 = jnp.float32

    def dot(a, b, dims):
        return lax.dot_general(a, b, (dims, ((), ())),
                               preferred_element_type=f32)

    def side(x, S_refs, C_ref, Ws, N):
        W1f, b1f, W2f, b2f, W1s, b1s, W2s, b2s = Ws
        ones = jnp.full((N, 1), 1.0, f32)
        Ms, diss = [], []
        for k in range(3):
            M = C_ref[k] * S_refs[k][...]
            deg = dot(M, ones, (((0,), (0,)))) + 1.0
            diss.append(lax.rsqrt(deg))
            Ms.append(M)

        def layer(h_in, W, b, M, dis):
            h = dot(h_in, W, ((1,), (1,)))
            v = dis * h
            agg = dot(M, v, ((0,), (0,)))
            out = dis * agg + (dis * dis) * h + b
            return jnp.maximum(out, 0.0)

        f1 = layer(x, W1f, b1f, Ms[0], diss[0])
        f2 = layer(f1, W2f, b2f, Ms[0], diss[0])
        s1 = layer(x, W1s, b1s, Ms[1], diss[1])
        s2 = layer(s1, W2s, b2s, Ms[1], diss[1])
        g1 = layer(x, W1s, b1s, Ms[2], diss[2])
        g2 = layer(s1, W2s, b2s, Ms[2], diss[2])
        return [f1, f2, s1, s2, g1, g2]

    def atten(feats, fc1W, fc1b, fc2W, fc2b, cw, cb, N):
        scale = 1.0 / (N * FM)
        a = jnp.concatenate(
            [(jnp.sum(f) * scale).reshape(1, 1) for f in feats], axis=1)
        a = jnp.maximum(dot(a, fc1W[...], ((1,), (1,))) + fc1b[...], 0.0)
        a = jax.nn.sigmoid(dot(a, fc2W[...], ((1,), (1,))) + fc2b[...])
        cwv = cw[...]
        out = cb[0, 0]
        for c in range(6):
            out = out + cwv[0, c] * jnp.maximum(a[0, c] * feats[c], 0.0)
        return out

    xw = (Wx1f[...], bx1f[...], Wx2f[...], bx2f[...],
          Wx1s[...], bx1s[...], Wx2s[...], bx2s[...])
    yw = (Wy1f[...], by1f[...], Wy2f[...], by2f[...],
          Wy1s[...], by1s[...], Wy2s[...], by2s[...])
    mf = side(xm[...], (Sm0, Sm1, Sm2), Cm, xw, NM)
    df = side(xd[...], (Sd0, Sd1, Sd2), Cd, yw, ND)
    x = atten(mf, fc1xW, fc1xb, fc2xW, fc2xb, cnnxw, cnnxb, NM)
    y = atten(df, fc1yW, fc1yb, fc2yW, fc2yb, cnnyw, cnnyb, ND)
    xp = jnp.concatenate(
        [x * linW[...], jnp.zeros((GR - NM, FM), f32)], axis=0)
    yp = jnp.concatenate([y, jnp.zeros((GC - ND, FM), f32)], axis=0)
    G_out[...] = dot(xp, yp, ((1,), (1,))) + linb[0, 0]


@functools.partial(
    pl.kernel,
    out_type=jax.ShapeDtypeStruct((P,), jnp.float32),
    mesh=_MESH,
    scratch_types=[
        pltpu.VMEM((P // 32,), jnp.int32),
        pltpu.VMEM((P // 32,), jnp.int32),
        pltpu.VMEM((4, 128), jnp.int32),
        pltpu.VMEM((4, 128), jnp.float32),
        pltpu.VMEM((P // 32,), jnp.float32),
        pltpu.SemaphoreType.DMA,
        pltpu.SemaphoreType.DMA,
    ],
)
def _sc_pair_gather(g_hbm, te_hbm, out_hbm, t0_v, t1_v, sidx_v, r_v, o_v,
                    sem, sem2):
    wid = lax.axis_index("s") * 2 + lax.axis_index("c")
    npp = P // 32
    base = wid * npp
    d0 = pltpu.async_copy(te_hbm.at[pl.ds(base, npp)], t0_v, sem)
    d1 = pltpu.async_copy(te_hbm.at[pl.ds(P + base, npp)], t1_v, sem)
    d0.wait()
    d1.wait()
    for v in range(npp // 16):
        i0 = t0_v[pl.ds(v * 16, 16)]
        i1 = t1_v[pl.ds(v * 16, 16)]
        sidx_v[v // 8, pl.ds((v % 8) * 16, 16)] = i0 * GC + i1
    gd = [pltpu.async_copy(g_hbm.at[sidx_v.at[j]], r_v.at[j], sem2)
          for j in range(npp // 128)]
    for d in gd:
        d.wait()
    for v in range(npp // 16):
        g = r_v[v // 8, pl.ds((v % 8) * 16, 16)]
        o_v[pl.ds(v * 16, 16)] = 1.0 / (1.0 + jnp.exp(-g))
    pltpu.sync_copy(o_v, out_hbm.at[pl.ds(base, npp)])


def kernel(mi_gua, mi_cos, mi_fun, di_gua, di_cos, di_sem, x_m, x_d,
           Wx1f, bx1f, Wx2f, bx2f, Wx1s, bx1s, Wx2s, bx2s,
           Wy1f, by1f, Wy2f, by2f, Wy1s, by1s, Wy2s, by2s,
           fc1x_W, fc1x_b, fc2x_W, fc2x_b, fc1y_W, fc1y_b, fc2y_W, fc2y_b,
           cnnx_w, cnnx_b, cnny_w, cnny_b, lin_W, lin_b,
           mi_gua_edges, mi_cos_edges, mi_fun_edges,
           di_gua_edges, di_cos_edges, di_sem_edges, train_edges):
    f32 = jnp.float32

    cm_raw, cd_raw = _sc_counts(
        mi_gua_edges.reshape(-1), mi_cos_edges.reshape(-1),
        mi_fun_edges.reshape(-1),
        di_gua_edges.reshape(-1), di_cos_edges.reshape(-1),
        di_sem_edges.reshape(-1))
    Cm = cm_raw.reshape(3, NM2P)[:, :NM * NM].reshape(3, NM, NM)
    Cd = cd_raw.reshape(3, ND2P)[:, :ND * ND].reshape(3, ND, ND)

    G = pl.pallas_call(
        _tc_dense_body,
        out_shape=jax.ShapeDtypeStruct((GR, GC), f32),
    )(mi_gua, mi_cos, mi_fun, Cm, di_gua, di_cos, di_sem, Cd, x_m, x_d,
      Wx1f, bx1f.reshape(1, FM), Wx2f, bx2f.reshape(1, FM),
      Wx1s, bx1s.reshape(1, FM), Wx2s, bx2s.reshape(1, FM),
      Wy1f, by1f.reshape(1, FM), Wy2f, by2f.reshape(1, FM),
      Wy1s, by1s.reshape(1, FM), Wy2s, by2s.reshape(1, FM),
      fc1x_W, fc1x_b.reshape(1, 30), fc2x_W, fc2x_b.reshape(1, 6),
      fc1y_W, fc1y_b.reshape(1, 30), fc2y_W, fc2y_b.reshape(1, 6),
      cnnx_w.reshape(1, 6), cnnx_b.reshape(1, 1),
      cnny_w.reshape(1, 6), cnny_b.reshape(1, 1),
      lin_W, lin_b.reshape(1, 1))

    te = train_edges.T.reshape(-1)
    return _sc_pair_gather(G.reshape(-1), te)


# speedup vs baseline: 31.6377x; 1.0192x over previous
import functools

import jax
import jax.numpy as jnp
from jax import lax
from jax.experimental import pallas as pl
from jax.experimental.pallas import tpu as pltpu
from jax.experimental.pallas import tpu_sc as plsc

FM = 128
NM = 215
ND = 110
EM = 8192
ED = 4096
P = 16384
NM2P = 46336
ND2P = 12160
GR = 216
GC = 128
GP = GR * GC

_MESH = plsc.VectorSubcoreMesh(core_axis_name="c", subcore_axis_name="s",
                               num_cores=2, num_subcores=16)


@functools.partial(
    pl.kernel,
    out_type=[
        jax.ShapeDtypeStruct((3 * NM2P,), jnp.float32),
        jax.ShapeDtypeStruct((3 * ND2P,), jnp.float32),
    ],
    mesh=_MESH,
    scratch_types=[
        pltpu.VMEM_SHARED((3 * NM2P,), jnp.float32),
        pltpu.VMEM((NM2P // 16,), jnp.float32),
        pltpu.VMEM((3 * NM2P // 16,), jnp.float32),
        pltpu.VMEM((1536,), jnp.float32),
        pltpu.VMEM((EM // 16,), jnp.int32),
        pltpu.VMEM((EM // 16,), jnp.int32),
        pltpu.VMEM((EM // 16,), jnp.int32),
        pltpu.VMEM((EM // 16,), jnp.int32),
        pltpu.VMEM((EM // 16,), jnp.int32),
        pltpu.VMEM((EM // 16,), jnp.int32),
        pltpu.VMEM((1536,), jnp.int32),
        pltpu.SemaphoreType.DMA,
        pltpu.SemaphoreType.DMA,
    ],
)
def _sc_counts(em0_hbm, em1_hbm, em2_hbm, ed0_hbm, ed1_hbm, ed2_hbm,
               outm_hbm, outd_hbm, cbig, zb, wb,
               ones_v, e0a, e0b, e1a, e1b, e2a, e2b, sidx_v, sem, zsem):
    cid = lax.axis_index("c")
    sid = lax.axis_index("s")
    ebufs = [(e0a, e0b), (e1a, e1b), (e2a, e2b)]

    def scatter_side(e_hbms, E, N, N2P):
        epp = E // 16
        nidx = 3 * epp
        base = sid * epp
        zslice = N2P // 16

        in_dmas = []
        for k in range(3):
            in_dmas.append(pltpu.async_copy(
                e_hbms[k].at[pl.ds(base, epp)],
                ebufs[k][0].at[pl.ds(0, epp)], sem))
            in_dmas.append(pltpu.async_copy(
                e_hbms[k].at[pl.ds(E + base, epp)],
                ebufs[k][1].at[pl.ds(0, epp)], sem))

        for t in range(zslice // 16):
            zb[pl.ds(t * 16, 16)] = jnp.zeros((16,), jnp.float32)
        for t in range(nidx // 16):
            ones_v[pl.ds(t * 16, 16)] = jnp.ones((16,), jnp.float32)
        zdmas = [pltpu.async_copy(
            zb.at[pl.ds(0, zslice)],
            cbig.at[pl.ds(k * N2P + sid * zslice, zslice)], zsem)
            for k in range(3)]

        for d in in_dmas:
            d.wait()
        for k in range(3):
            for v in range(epp // 16):
                i0 = ebufs[k][0][pl.ds(v * 16, 16)]
                i1 = ebufs[k][1][pl.ds(v * 16, 16)]
                sidx_v[pl.ds(k * epp + v * 16, 16)] = i0 * N + i1 + k * N2P
        for d in zdmas:
            d.wait()
        plsc.subcore_barrier()

        pltpu.sync_copy(ones_v.at[pl.ds(0, nidx)],
                        cbig.at[sidx_v.at[pl.ds(0, nidx)]],
                        add=True)

    @pl.when(cid == 0)
    def _():
        scatter_side([em0_hbm, em1_hbm, em2_hbm], EM, NM, NM2P)

    @pl.when(cid == 1)
    def _():
        scatter_side([ed0_hbm, ed1_hbm, ed2_hbm], ED, ND, ND2P)

    plsc.subcore_barrier()

    def write_out(out_hbm, OUTP):
        oslice = 3 * OUTP // 16
        pltpu.sync_copy(cbig.at[pl.ds(sid * oslice, oslice)],
                        wb.at[pl.ds(0, oslice)])
        pltpu.async_copy(wb.at[pl.ds(0, oslice)],
                         out_hbm.at[pl.ds(sid * oslice, oslice)], sem).wait()

    @pl.when(cid == 0)
    def _():
        write_out(outm_hbm, NM2P)

    @pl.when(cid == 1)
    def _():
        write_out(outd_hbm, ND2P)


def _tc_dense_body(Sm0, Sm1, Sm2, Cm, Sd0, Sd1, Sd2, Cd, xm, xd,
                   Wx1f, bx1f, Wx2f, bx2f, Wx1s, bx1s, Wx2s, bx2s,
                   Wy1f, by1f, Wy2f, by2f, Wy1s, by1s, Wy2s, by2s,
                   fc1xW, fc1xb, fc2xW, fc2xb, fc1yW, fc1yb, fc2yW, fc2yb,
                   cnnxw, cnnxb, cnnyw, cnnyb, linW, linb, G_out):
    f32 = jnp.float32

    def dot(a, b, dims):
        return lax.dot_general(a, b, (dims, ((), ())),
                               preferred_element_type=f32)

    def side(x, S_refs, C_ref, Ws, N):
        W1f, b1f, W2f, b2f, W1s, b1s, W2s, b2s = Ws
        ones = jnp.full((N, 1), 1.0, f32)
        Ms, diss = [], []
        for k in range(3):
            M = C_ref[k] * S_refs[k][...]
            deg = dot(M, ones, (((0,), (0,)))) + 1.0
            diss.append(lax.rsqrt(deg))
            Ms.append(M)

        def layer(h_in, W, b, M, dis):
            h = dot(h_in, W, ((1,), (1,)))
            v = dis * h
            agg = dot(M, v, ((0,), (0,)))
            out = dis * agg + (dis * dis) * h + b
            return jnp.maximum(out, 0.0)

        f1 = layer(x, W1f, b1f, Ms[0], diss[0])
        f2 = layer(f1, W2f, b2f, Ms[0], diss[0])
        s1 = layer(x, W1s, b1s, Ms[1], diss[1])
        s2 = layer(s1, W2s, b2s, Ms[1], diss[1])
        g1 = layer(x, W1s, b1s, Ms[2], diss[2])
        g2 = layer(s1, W2s, b2s, Ms[2], diss[2])
        return [f1, f2, s1, s2, g1, g2]

    def atten(feats, fc1W, fc1b, fc2W, fc2b, cw, cb, N):
        scale = 1.0 / (N * FM)
        a = jnp.concatenate(
            [(jnp.sum(f) * scale).reshape(1, 1) for f in feats], axis=1)
        a = jnp.maximum(dot(a, fc1W[...], ((1,), (1,))) + fc1b[...], 0.0)
        a = jax.nn.sigmoid(dot(a, fc2W[...], ((1,), (1,))) + fc2b[...])
        cwv = cw[...]
        out = cb[0, 0]
        for c in range(6):
            out = out + cwv[0, c] * jnp.maximum(a[0, c] * feats[c], 0.0)
        return out

    xw = (Wx1f[...], bx1f[...], Wx2f[...], bx2f[...],
          Wx1s[...], bx1s[...], Wx2s[...], bx2s[...])
    yw = (Wy1f[...], by1f[...], Wy2f[...], by2f[...],
          Wy1s[...], by1s[...], Wy2s[...], by2s[...])
    mf = side(xm[...], (Sm0, Sm1, Sm2), Cm, xw, NM)
    df = side(xd[...], (Sd0, Sd1, Sd2), Cd, yw, ND)
    x = atten(mf, fc1xW, fc1xb, fc2xW, fc2xb, cnnxw, cnnxb, NM)
    y = atten(df, fc1yW, fc1yb, fc2yW, fc2yb, cnnyw, cnnyb, ND)
    xp = jnp.concatenate(
        [x * linW[...], jnp.zeros((GR - NM, FM), f32)], axis=0)
    yp = jnp.concatenate([y, jnp.zeros((GC - ND, FM), f32)], axis=0)
    G_out[...] = dot(xp, yp, ((1,), (1,))) + linb[0, 0]


@functools.partial(
    pl.kernel,
    out_type=jax.ShapeDtypeStruct((P,), jnp.float32),
    mesh=_MESH,
    scratch_types=[
        pltpu.VMEM((P // 32,), jnp.int32),
        pltpu.VMEM((P // 32,), jnp.int32),
        pltpu.VMEM((P // 32,), jnp.int32),
        pltpu.VMEM((P // 32,), jnp.float32),
        pltpu.VMEM((P // 32,), jnp.float32),
        pltpu.SemaphoreType.DMA,
        pltpu.SemaphoreType.DMA,
    ],
)
def _sc_pair_gather(g_hbm, te_hbm, out_hbm, t0_v, t1_v, sidx_v, r_v, o_v,
                    sem, sem2):
    wid = lax.axis_index("s") * 2 + lax.axis_index("c")
    npp = P // 32
    base = wid * npp
    d0 = pltpu.async_copy(te_hbm.at[pl.ds(base, npp)], t0_v, sem)
    d1 = pltpu.async_copy(te_hbm.at[pl.ds(P + base, npp)], t1_v, sem)
    d0.wait()
    d1.wait()
    for v in range(npp // 16):
        i0 = t0_v[pl.ds(v * 16, 16)]
        i1 = t1_v[pl.ds(v * 16, 16)]
        sidx_v[pl.ds(v * 16, 16)] = i0 * GC + i1
    pltpu.async_copy(g_hbm.at[sidx_v], r_v, sem2).wait()
    for v in range(npp // 16):
        g = r_v[pl.ds(v * 16, 16)]
        o_v[pl.ds(v * 16, 16)] = 1.0 / (1.0 + jnp.exp(-g))
    pltpu.sync_copy(o_v, out_hbm.at[pl.ds(base, npp)])


def kernel(mi_gua, mi_cos, mi_fun, di_gua, di_cos, di_sem, x_m, x_d,
           Wx1f, bx1f, Wx2f, bx2f, Wx1s, bx1s, Wx2s, bx2s,
           Wy1f, by1f, Wy2f, by2f, Wy1s, by1s, Wy2s, by2s,
           fc1x_W, fc1x_b, fc2x_W, fc2x_b, fc1y_W, fc1y_b, fc2y_W, fc2y_b,
           cnnx_w, cnnx_b, cnny_w, cnny_b, lin_W, lin_b,
           mi_gua_edges, mi_cos_edges, mi_fun_edges,
           di_gua_edges, di_cos_edges, di_sem_edges, train_edges):
    f32 = jnp.float32

    cm_raw, cd_raw = _sc_counts(
        mi_gua_edges.reshape(-1), mi_cos_edges.reshape(-1),
        mi_fun_edges.reshape(-1),
        di_gua_edges.reshape(-1), di_cos_edges.reshape(-1),
        di_sem_edges.reshape(-1))
    Cm = cm_raw.reshape(3, NM2P)[:, :NM * NM].reshape(3, NM, NM)
    Cd = cd_raw.reshape(3, ND2P)[:, :ND * ND].reshape(3, ND, ND)

    G = pl.pallas_call(
        _tc_dense_body,
        out_shape=jax.ShapeDtypeStruct((GR, GC), f32),
    )(mi_gua, mi_cos, mi_fun, Cm, di_gua, di_cos, di_sem, Cd, x_m, x_d,
      Wx1f, bx1f.reshape(1, FM), Wx2f, bx2f.reshape(1, FM),
      Wx1s, bx1s.reshape(1, FM), Wx2s, bx2s.reshape(1, FM),
      Wy1f, by1f.reshape(1, FM), Wy2f, by2f.reshape(1, FM),
      Wy1s, by1s.reshape(1, FM), Wy2s, by2s.reshape(1, FM),
      fc1x_W, fc1x_b.reshape(1, 30), fc2x_W, fc2x_b.reshape(1, 6),
      fc1y_W, fc1y_b.reshape(1, 30), fc2y_W, fc2y_b.reshape(1, 6),
      cnnx_w.reshape(1, 6), cnnx_b.reshape(1, 1),
      cnny_w.reshape(1, 6), cnny_b.reshape(1, 1),
      lin_W, lin_b.reshape(1, 1))

    te = train_edges.T.reshape(-1)
    return _sc_pair_gather(G.reshape(-1), te)


# speedup vs baseline: 32.7487x; 1.0351x over previous
import functools

import jax
import jax.numpy as jnp
from jax import lax
from jax.experimental import pallas as pl
from jax.experimental.pallas import tpu as pltpu
from jax.experimental.pallas import tpu_sc as plsc

FM = 128
NM = 215
ND = 110
EM = 8192
ED = 4096
P = 16384
NM2P = 46336
ND2P = 12160
GR = 216
GC = 128
GP = GR * GC

_MESH = plsc.VectorSubcoreMesh(core_axis_name="c", subcore_axis_name="s",
                               num_cores=2, num_subcores=16)


@functools.partial(
    pl.kernel,
    out_type=[
        jax.ShapeDtypeStruct((3 * NM2P,), jnp.float32),
        jax.ShapeDtypeStruct((3 * ND2P,), jnp.float32),
    ],
    mesh=_MESH,
    scratch_types=[
        pltpu.VMEM_SHARED((3 * NM2P,), jnp.float32),
        pltpu.VMEM((NM2P // 16,), jnp.float32),
        pltpu.VMEM((3 * NM2P // 16,), jnp.float32),
        pltpu.VMEM((1536,), jnp.float32),
        pltpu.VMEM((EM // 16,), jnp.int32),
        pltpu.VMEM((EM // 16,), jnp.int32),
        pltpu.VMEM((EM // 16,), jnp.int32),
        pltpu.VMEM((EM // 16,), jnp.int32),
        pltpu.VMEM((EM // 16,), jnp.int32),
        pltpu.VMEM((EM // 16,), jnp.int32),
        pltpu.VMEM((1536,), jnp.int32),
        pltpu.SemaphoreType.DMA,
        pltpu.SemaphoreType.DMA,
    ],
)
def _sc_counts(em0_hbm, em1_hbm, em2_hbm, ed0_hbm, ed1_hbm, ed2_hbm,
               outm_hbm, outd_hbm, cbig, zb, wb,
               ones_v, e0a, e0b, e1a, e1b, e2a, e2b, sidx_v, sem, zsem):
    cid = lax.axis_index("c")
    sid = lax.axis_index("s")
    ebufs = [(e0a, e0b), (e1a, e1b), (e2a, e2b)]

    def scatter_side(e_hbms, E, N, N2P):
        epp = E // 16
        nidx = 3 * epp
        base = sid * epp
        zslice = N2P // 16

        in_dmas = []
        for k in range(3):
            in_dmas.append(pltpu.async_copy(
                e_hbms[k].at[pl.ds(base, epp)],
                ebufs[k][0].at[pl.ds(0, epp)], sem))
            in_dmas.append(pltpu.async_copy(
                e_hbms[k].at[pl.ds(E + base, epp)],
                ebufs[k][1].at[pl.ds(0, epp)], sem))

        for t in range(zslice // 16):
            zb[pl.ds(t * 16, 16)] = jnp.zeros((16,), jnp.float32)
        for t in range(nidx // 16):
            ones_v[pl.ds(t * 16, 16)] = jnp.ones((16,), jnp.float32)
        zdmas = [pltpu.async_copy(
            zb.at[pl.ds(0, zslice)],
            cbig.at[pl.ds(k * N2P + sid * zslice, zslice)], zsem)
            for k in range(3)]

        for d in in_dmas:
            d.wait()
        for k in range(3):
            for v in range(epp // 16):
                i0 = ebufs[k][0][pl.ds(v * 16, 16)]
                i1 = ebufs[k][1][pl.ds(v * 16, 16)]
                sidx_v[pl.ds(k * epp + v * 16, 16)] = i0 * N + i1 + k * N2P
        for d in zdmas:
            d.wait()
        plsc.subcore_barrier()

        pltpu.sync_copy(ones_v.at[pl.ds(0, nidx)],
                        cbig.at[sidx_v.at[pl.ds(0, nidx)]],
                        add=True)

    @pl.when(cid == 0)
    def _():
        scatter_side([em0_hbm, em1_hbm, em2_hbm], EM, NM, NM2P)

    @pl.when(cid == 1)
    def _():
        scatter_side([ed0_hbm, ed1_hbm, ed2_hbm], ED, ND, ND2P)

    plsc.subcore_barrier()

    def write_out(out_hbm, OUTP):
        oslice = 3 * OUTP // 16
        pltpu.sync_copy(cbig.at[pl.ds(sid * oslice, oslice)],
                        wb.at[pl.ds(0, oslice)])
        pltpu.async_copy(wb.at[pl.ds(0, oslice)],
                         out_hbm.at[pl.ds(sid * oslice, oslice)], sem).wait()

    @pl.when(cid == 0)
    def _():
        write_out(outm_hbm, NM2P)

    @pl.when(cid == 1)
    def _():
        write_out(outd_hbm, ND2P)


def _tc_dense_body(Sm0, Sm1, Sm2, Cm, Sd0, Sd1, Sd2, Cd, xm, xd,
                   Wx1f, bx1f, Wx2f, bx2f, Wx1s, bx1s, Wx2s, bx2s,
                   Wy1f, by1f, Wy2f, by2f, Wy1s, by1s, Wy2s, by2s,
                   fc1xW, fc1xb, fc2xW, fc2xb, fc1yW, fc1yb, fc2yW, fc2yb,
                   cnnxw, cnnxb, cnnyw, cnnyb, linW, linb, G_out):
    f32 = jnp.float32

    def dot(a, b, dims):
        return lax.dot_general(a, b, (dims, ((), ())),
                               preferred_element_type=f32)

    def side(x, S_refs, C_ref, Ws, N):
        W1f, b1f, W2f, b2f, W1s, b1s, W2s, b2s = Ws
        ones = jnp.full((N, 1), 1.0, f32)
        Ms, diss = [], []
        for k in range(3):
            M = C_ref[k] * S_refs[k][...]
            deg = dot(M, ones, (((0,), (0,)))) + 1.0
            diss.append(lax.rsqrt(deg))
            Ms.append(M)

        def layer(h_in, W, b, M, dis):
            h = dot(h_in, W, ((1,), (1,)))
            v = dis * h
            agg = dot(M, v, ((0,), (0,)))
            out = dis * agg + (dis * dis) * h + b
            return jnp.maximum(out, 0.0)

        f1 = layer(x, W1f, b1f, Ms[0], diss[0])
        f2 = layer(f1, W2f, b2f, Ms[0], diss[0])
        s1 = layer(x, W1s, b1s, Ms[1], diss[1])
        s2 = layer(s1, W2s, b2s, Ms[1], diss[1])
        g1 = layer(x, W1s, b1s, Ms[2], diss[2])
        g2 = layer(s1, W2s, b2s, Ms[2], diss[2])
        return [f1, f2, s1, s2, g1, g2]

    def atten(feats, fc1W, fc1b, fc2W, fc2b, cw, cb, N):
        scale = 1.0 / (N * FM)
        a = jnp.concatenate(
            [(jnp.sum(f) * scale).reshape(1, 1) for f in feats], axis=1)
        a = jnp.maximum(dot(a, fc1W[...], ((1,), (1,))) + fc1b[...], 0.0)
        a = jax.nn.sigmoid(dot(a, fc2W[...], ((1,), (1,))) + fc2b[...])
        cwv = cw[...]
        out = cb[0, 0]
        for c in range(6):
            out = out + cwv[0, c] * jnp.maximum(a[0, c] * feats[c], 0.0)
        return out

    xw = (Wx1f[...], bx1f[...], Wx2f[...], bx2f[...],
          Wx1s[...], bx1s[...], Wx2s[...], bx2s[...])
    yw = (Wy1f[...], by1f[...], Wy2f[...], by2f[...],
          Wy1s[...], by1s[...], Wy2s[...], by2s[...])
    mf = side(xm[...], (Sm0, Sm1, Sm2), Cm, xw, NM)
    df = side(xd[...], (Sd0, Sd1, Sd2), Cd, yw, ND)
    x = atten(mf, fc1xW, fc1xb, fc2xW, fc2xb, cnnxw, cnnxb, NM)
    y = atten(df, fc1yW, fc1yb, fc2yW, fc2yb, cnnyw, cnnyb, ND)
    xp = jnp.concatenate(
        [x * linW[...], jnp.zeros((GR - NM, FM), f32)], axis=0)
    yp = jnp.concatenate([y, jnp.zeros((GC - ND, FM), f32)], axis=0)
    G_out[...] = dot(xp, yp, ((1,), (1,))) + linb[0, 0]


@functools.partial(
    pl.kernel,
    out_type=jax.ShapeDtypeStruct((P,), jnp.float32),
    mesh=_MESH,
    scratch_types=[
        pltpu.VMEM_SHARED((GP,), jnp.float32),
        pltpu.VMEM((GP // 16,), jnp.float32),
        pltpu.VMEM((P // 32,), jnp.int32),
        pltpu.VMEM((P // 32,), jnp.int32),
        pltpu.VMEM((P // 32,), jnp.int32),
        pltpu.VMEM((P // 32,), jnp.float32),
        pltpu.VMEM((P // 32,), jnp.float32),
        pltpu.SemaphoreType.DMA,
        pltpu.SemaphoreType.DMA,
    ],
)
def _sc_pair_gather(g_hbm, te_hbm, out_hbm, g_sp, gb, t0_v, t1_v, sidx_v,
                    r_v, o_v, sem, sem2):
    sid = lax.axis_index("s")
    wid = sid * 2 + lax.axis_index("c")
    npp = P // 32
    base = wid * npp
    gsl = GP // 16
    gdma = pltpu.async_copy(g_hbm.at[pl.ds(sid * gsl, gsl)], gb, sem2)
    d0 = pltpu.async_copy(te_hbm.at[pl.ds(base, npp)], t0_v, sem)
    d1 = pltpu.async_copy(te_hbm.at[pl.ds(P + base, npp)], t1_v, sem)
    d0.wait()
    d1.wait()
    for v in range(npp // 16):
        i0 = t0_v[pl.ds(v * 16, 16)]
        i1 = t1_v[pl.ds(v * 16, 16)]
        sidx_v[pl.ds(v * 16, 16)] = i0 * GC + i1
    gdma.wait()
    pltpu.sync_copy(gb, g_sp.at[pl.ds(sid * gsl, gsl)])
    plsc.subcore_barrier()
    pltpu.sync_copy(g_sp.at[sidx_v], r_v)
    for v in range(npp // 16):
        g = r_v[pl.ds(v * 16, 16)]
        o_v[pl.ds(v * 16, 16)] = 1.0 / (1.0 + jnp.exp(-g))
    pltpu.sync_copy(o_v, out_hbm.at[pl.ds(base, npp)])


def kernel(mi_gua, mi_cos, mi_fun, di_gua, di_cos, di_sem, x_m, x_d,
           Wx1f, bx1f, Wx2f, bx2f, Wx1s, bx1s, Wx2s, bx2s,
           Wy1f, by1f, Wy2f, by2f, Wy1s, by1s, Wy2s, by2s,
           fc1x_W, fc1x_b, fc2x_W, fc2x_b, fc1y_W, fc1y_b, fc2y_W, fc2y_b,
           cnnx_w, cnnx_b, cnny_w, cnny_b, lin_W, lin_b,
           mi_gua_edges, mi_cos_edges, mi_fun_edges,
           di_gua_edges, di_cos_edges, di_sem_edges, train_edges):
    f32 = jnp.float32

    cm_raw, cd_raw = _sc_counts(
        mi_gua_edges.reshape(-1), mi_cos_edges.reshape(-1),
        mi_fun_edges.reshape(-1),
        di_gua_edges.reshape(-1), di_cos_edges.reshape(-1),
        di_sem_edges.reshape(-1))
    Cm = cm_raw.reshape(3, NM2P)[:, :NM * NM].reshape(3, NM, NM)
    Cd = cd_raw.reshape(3, ND2P)[:, :ND * ND].reshape(3, ND, ND)

    G = pl.pallas_call(
        _tc_dense_body,
        out_shape=jax.ShapeDtypeStruct((GR, GC), f32),
    )(mi_gua, mi_cos, mi_fun, Cm, di_gua, di_cos, di_sem, Cd, x_m, x_d,
      Wx1f, bx1f.reshape(1, FM), Wx2f, bx2f.reshape(1, FM),
      Wx1s, bx1s.reshape(1, FM), Wx2s, bx2s.reshape(1, FM),
      Wy1f, by1f.reshape(1, FM), Wy2f, by2f.reshape(1, FM),
      Wy1s, by1s.reshape(1, FM), Wy2s, by2s.reshape(1, FM),
      fc1x_W, fc1x_b.reshape(1, 30), fc2x_W, fc2x_b.reshape(1, 6),
      fc1y_W, fc1y_b.reshape(1, 30), fc2y_W, fc2y_b.reshape(1, 6),
      cnnx_w.reshape(1, 6), cnnx_b.reshape(1, 1),
      cnny_w.reshape(1, 6), cnny_b.reshape(1, 1),
      lin_W, lin_b.reshape(1, 1))

    te = train_edges.T.reshape(-1)
    return _sc_pair_gather(G.reshape(-1), te)


# speedup vs baseline: 32.7857x; 1.0011x over previous
import functools

import jax
import jax.numpy as jnp
from jax import lax
from jax.experimental import pallas as pl
from jax.experimental.pallas import tpu as pltpu
from jax.experimental.pallas import tpu_sc as plsc

FM = 128
NM = 215
ND = 110
EM = 8192
ED = 4096
P = 16384
NM2P = 46336
ND2P = 12160
GR = 216
GC = 128
GP = GR * GC

_MESH = plsc.VectorSubcoreMesh(core_axis_name="c", subcore_axis_name="s",
                               num_cores=2, num_subcores=16)


@functools.partial(
    pl.kernel,
    out_type=[
        jax.ShapeDtypeStruct((3 * NM2P,), jnp.float32),
        jax.ShapeDtypeStruct((3 * ND2P,), jnp.float32),
    ],
    mesh=_MESH,
    scratch_types=[
        pltpu.VMEM_SHARED((3 * NM2P,), jnp.float32),
        pltpu.VMEM((NM2P // 16,), jnp.float32),
        pltpu.VMEM((3 * NM2P // 16,), jnp.float32),
        pltpu.VMEM((1536,), jnp.float32),
        pltpu.VMEM((EM // 16,), jnp.int32),
        pltpu.VMEM((EM // 16,), jnp.int32),
        pltpu.VMEM((EM // 16,), jnp.int32),
        pltpu.VMEM((EM // 16,), jnp.int32),
        pltpu.VMEM((EM // 16,), jnp.int32),
        pltpu.VMEM((EM // 16,), jnp.int32),
        pltpu.VMEM((1536,), jnp.int32),
        pltpu.SemaphoreType.DMA,
        pltpu.SemaphoreType.DMA,
    ],
)
def _sc_counts(em0_hbm, em1_hbm, em2_hbm, ed0_hbm, ed1_hbm, ed2_hbm,
               outm_hbm, outd_hbm, cbig, zb, wb,
               ones_v, e0a, e0b, e1a, e1b, e2a, e2b, sidx_v, sem, zsem):
    cid = lax.axis_index("c")
    sid = lax.axis_index("s")
    ebufs = [(e0a, e0b), (e1a, e1b), (e2a, e2b)]

    def scatter_side(e_hbms, E, N, N2P):
        epp = E // 16
        nidx = 3 * epp
        base = sid * epp
        zslice = N2P // 16

        in_dmas = []
        for k in range(3):
            in_dmas.append(pltpu.async_copy(
                e_hbms[k].at[pl.ds(base, epp)],
                ebufs[k][0].at[pl.ds(0, epp)], sem))
            in_dmas.append(pltpu.async_copy(
                e_hbms[k].at[pl.ds(E + base, epp)],
                ebufs[k][1].at[pl.ds(0, epp)], sem))

        for t in range(zslice // 16):
            zb[pl.ds(t * 16, 16)] = jnp.zeros((16,), jnp.float32)
        for t in range(nidx // 16):
            ones_v[pl.ds(t * 16, 16)] = jnp.ones((16,), jnp.float32)
        zdmas = [pltpu.async_copy(
            zb.at[pl.ds(0, zslice)],
            cbig.at[pl.ds(k * N2P + sid * zslice, zslice)], zsem)
            for k in range(3)]

        for d in in_dmas:
            d.wait()
        for k in range(3):
            for v in range(epp // 16):
                i0 = ebufs[k][0][pl.ds(v * 16, 16)]
                i1 = ebufs[k][1][pl.ds(v * 16, 16)]
                sidx_v[pl.ds(k * epp + v * 16, 16)] = i0 * N + i1 + k * N2P
        for d in zdmas:
            d.wait()
        plsc.subcore_barrier()

        pltpu.sync_copy(ones_v.at[pl.ds(0, nidx)],
                        cbig.at[sidx_v.at[pl.ds(0, nidx)]],
                        add=True)

    @pl.when(cid == 0)
    def _():
        scatter_side([em0_hbm, em1_hbm, em2_hbm], EM, NM, NM2P)

    @pl.when(cid == 1)
    def _():
        scatter_side([ed0_hbm, ed1_hbm, ed2_hbm], ED, ND, ND2P)

    plsc.subcore_barrier()

    def write_out(out_hbm, OUTP):
        oslice = 3 * OUTP // 16
        half = (oslice // 16) * 8
        out_dmas = []
        for h, hlen in ((0, half), (half, oslice - half)):
            pltpu.sync_copy(cbig.at[pl.ds(sid * oslice + h, hlen)],
                            wb.at[pl.ds(h, hlen)])
            out_dmas.append(pltpu.async_copy(
                wb.at[pl.ds(h, hlen)],
                out_hbm.at[pl.ds(sid * oslice + h, hlen)], sem))
        for d in out_dmas:
            d.wait()

    @pl.when(cid == 0)
    def _():
        write_out(outm_hbm, NM2P)

    @pl.when(cid == 1)
    def _():
        write_out(outd_hbm, ND2P)


def _tc_dense_body(Sm0, Sm1, Sm2, Cm, Sd0, Sd1, Sd2, Cd, xm, xd,
                   Wx1f, bx1f, Wx2f, bx2f, Wx1s, bx1s, Wx2s, bx2s,
                   Wy1f, by1f, Wy2f, by2f, Wy1s, by1s, Wy2s, by2s,
                   fc1xW, fc1xb, fc2xW, fc2xb, fc1yW, fc1yb, fc2yW, fc2yb,
                   cnnxw, cnnxb, cnnyw, cnnyb, linW, linb, G_out):
    f32 = jnp.float32

    def dot(a, b, dims):
        return lax.dot_general(a, b, (dims, ((), ())),
                               preferred_element_type=f32)

    def side(x, S_refs, C_ref, Ws, N):
        W1f, b1f, W2f, b2f, W1s, b1s, W2s, b2s = Ws
        ones = jnp.full((N, 1), 1.0, f32)
        Ms, diss = [], []
        for k in range(3):
            M = C_ref[k] * S_refs[k][...]
            deg = dot(M, ones, (((0,), (0,)))) + 1.0
            diss.append(lax.rsqrt(deg))
            Ms.append(M)

        def layer(h_in, W, b, M, dis):
            h = dot(h_in, W, ((1,), (1,)))
            v = dis * h
            agg = dot(M, v, ((0,), (0,)))
            out = dis * agg + (dis * dis) * h + b
            return jnp.maximum(out, 0.0)

        f1 = layer(x, W1f, b1f, Ms[0], diss[0])
        f2 = layer(f1, W2f, b2f, Ms[0], diss[0])
        s1 = layer(x, W1s, b1s, Ms[1], diss[1])
        s2 = layer(s1, W2s, b2s, Ms[1], diss[1])
        g1 = layer(x, W1s, b1s, Ms[2], diss[2])
        g2 = layer(s1, W2s, b2s, Ms[2], diss[2])
        return [f1, f2, s1, s2, g1, g2]

    def atten(feats, fc1W, fc1b, fc2W, fc2b, cw, cb, N):
        scale = 1.0 / (N * FM)
        a = jnp.concatenate(
            [(jnp.sum(f) * scale).reshape(1, 1) for f in feats], axis=1)
        a = jnp.maximum(dot(a, fc1W[...], ((1,), (1,))) + fc1b[...], 0.0)
        a = jax.nn.sigmoid(dot(a, fc2W[...], ((1,), (1,))) + fc2b[...])
        cwv = cw[...]
        out = cb[0, 0]
        for c in range(6):
            out = out + cwv[0, c] * jnp.maximum(a[0, c] * feats[c], 0.0)
        return out

    xw = (Wx1f[...], bx1f[...], Wx2f[...], bx2f[...],
          Wx1s[...], bx1s[...], Wx2s[...], bx2s[...])
    yw = (Wy1f[...], by1f[...], Wy2f[...], by2f[...],
          Wy1s[...], by1s[...], Wy2s[...], by2s[...])
    mf = side(xm[...], (Sm0, Sm1, Sm2), Cm, xw, NM)
    df = side(xd[...], (Sd0, Sd1, Sd2), Cd, yw, ND)
    x = atten(mf, fc1xW, fc1xb, fc2xW, fc2xb, cnnxw, cnnxb, NM)
    y = atten(df, fc1yW, fc1yb, fc2yW, fc2yb, cnnyw, cnnyb, ND)
    xp = jnp.concatenate(
        [x * linW[...], jnp.zeros((GR - NM, FM), f32)], axis=0)
    yp = jnp.concatenate([y, jnp.zeros((GC - ND, FM), f32)], axis=0)
    G_out[...] = dot(xp, yp, ((1,), (1,))) + linb[0, 0]


@functools.partial(
    pl.kernel,
    out_type=jax.ShapeDtypeStruct((P,), jnp.float32),
    mesh=_MESH,
    scratch_types=[
        pltpu.VMEM_SHARED((GP,), jnp.float32),
        pltpu.VMEM((GP // 16,), jnp.float32),
        pltpu.VMEM((P // 32,), jnp.int32),
        pltpu.VMEM((P // 32,), jnp.int32),
        pltpu.VMEM((P // 32,), jnp.int32),
        pltpu.VMEM((P // 32,), jnp.float32),
        pltpu.VMEM((P // 32,), jnp.float32),
        pltpu.SemaphoreType.DMA,
        pltpu.SemaphoreType.DMA,
    ],
)
def _sc_pair_gather(g_hbm, te_hbm, out_hbm, g_sp, gb, t0_v, t1_v, sidx_v,
                    r_v, o_v, sem, sem2):
    sid = lax.axis_index("s")
    wid = sid * 2 + lax.axis_index("c")
    npp = P // 32
    base = wid * npp
    gsl = GP // 16
    gdma = pltpu.async_copy(g_hbm.at[pl.ds(sid * gsl, gsl)], gb, sem2)
    d0 = pltpu.async_copy(te_hbm.at[pl.ds(base, npp)], t0_v, sem)
    d1 = pltpu.async_copy(te_hbm.at[pl.ds(P + base, npp)], t1_v, sem)
    d0.wait()
    d1.wait()
    for v in range(npp // 16):
        i0 = t0_v[pl.ds(v * 16, 16)]
        i1 = t1_v[pl.ds(v * 16, 16)]
        sidx_v[pl.ds(v * 16, 16)] = i0 * GC + i1
    gdma.wait()
    pltpu.sync_copy(gb, g_sp.at[pl.ds(sid * gsl, gsl)])
    plsc.subcore_barrier()
    pltpu.sync_copy(g_sp.at[sidx_v], r_v)
    for v in range(npp // 16):
        g = r_v[pl.ds(v * 16, 16)]
        o_v[pl.ds(v * 16, 16)] = 1.0 / (1.0 + jnp.exp(-g))
    pltpu.sync_copy(o_v, out_hbm.at[pl.ds(base, npp)])


def kernel(mi_gua, mi_cos, mi_fun, di_gua, di_cos, di_sem, x_m, x_d,
           Wx1f, bx1f, Wx2f, bx2f, Wx1s, bx1s, Wx2s, bx2s,
           Wy1f, by1f, Wy2f, by2f, Wy1s, by1s, Wy2s, by2s,
           fc1x_W, fc1x_b, fc2x_W, fc2x_b, fc1y_W, fc1y_b, fc2y_W, fc2y_b,
           cnnx_w, cnnx_b, cnny_w, cnny_b, lin_W, lin_b,
           mi_gua_edges, mi_cos_edges, mi_fun_edges,
           di_gua_edges, di_cos_edges, di_sem_edges, train_edges):
    f32 = jnp.float32

    cm_raw, cd_raw = _sc_counts(
        mi_gua_edges.reshape(-1), mi_cos_edges.reshape(-1),
        mi_fun_edges.reshape(-1),
        di_gua_edges.reshape(-1), di_cos_edges.reshape(-1),
        di_sem_edges.reshape(-1))
    Cm = cm_raw.reshape(3, NM2P)[:, :NM * NM].reshape(3, NM, NM)
    Cd = cd_raw.reshape(3, ND2P)[:, :ND * ND].reshape(3, ND, ND)

    G = pl.pallas_call(
        _tc_dense_body,
        out_shape=jax.ShapeDtypeStruct((GR, GC), f32),
    )(mi_gua, mi_cos, mi_fun, Cm, di_gua, di_cos, di_sem, Cd, x_m, x_d,
      Wx1f, bx1f.reshape(1, FM), Wx2f, bx2f.reshape(1, FM),
      Wx1s, bx1s.reshape(1, FM), Wx2s, bx2s.reshape(1, FM),
      Wy1f, by1f.reshape(1, FM), Wy2f, by2f.reshape(1, FM),
      Wy1s, by1s.reshape(1, FM), Wy2s, by2s.reshape(1, FM),
      fc1x_W, fc1x_b.reshape(1, 30), fc2x_W, fc2x_b.reshape(1, 6),
      fc1y_W, fc1y_b.reshape(1, 30), fc2y_W, fc2y_b.reshape(1, 6),
      cnnx_w.reshape(1, 6), cnnx_b.reshape(1, 1),
      cnny_w.reshape(1, 6), cnny_b.reshape(1, 1),
      lin_W, lin_b.reshape(1, 1))

    te = train_edges.T.reshape(-1)
    return _sc_pair_gather(G.reshape(-1), te)
